# Initial kernel scaffold; baseline (speedup 1.0000x reference)
import jax
import jax.numpy as jnp
from jax.experimental import pallas as pl


def kernel(x, edge_index, nlabel, Wn0, Ws0, b0, Wn1, Ws1, b1, Wn2, Ws2, b2, Wn3, Ws3, b3, lin1_W, lin1_b, lin2_W, lin2_b):
    raise NotImplementedError("write your pallas kernel here")


# speedup vs baseline: 6.3839x; 6.3839x over previous
import functools

import jax
import jax.numpy as jnp
from jax import lax
from jax.experimental import pallas as pl
from jax.experimental.pallas import tpu as pltpu
from jax.experimental.pallas import tpu_sc as plsc

_NC = 2
_NS = 16
_NW = _NC * _NS
_CHUNK = 128
_NPAD = 10240
_RPT = _NPAD // _NS


def _seg_sum_body(ncols, cpt, p_hbm, src_hbm, dst_hbm, z_hbm, out_hbm,
                  src_v, dst_v, rows_v, agg_sh):
    c = lax.axis_index("c")
    s = lax.axis_index("s")
    wid = s * _NC + c
    pltpu.sync_copy(z_hbm, agg_sh.at[pl.ds(s * _RPT, _RPT)])
    pltpu.sync_copy(src_hbm.at[pl.ds(wid * cpt, cpt)], src_v)
    pltpu.sync_copy(dst_hbm.at[pl.ds(wid * cpt, cpt)], dst_v)
    plsc.subcore_barrier()

    @pl.loop(0, cpt)
    def _(j):
        pltpu.sync_copy(p_hbm.at[src_v.at[j]], rows_v)
        pltpu.sync_copy(rows_v, agg_sh.at[dst_v.at[j]], add=True)

    plsc.subcore_barrier()
    pltpu.sync_copy(agg_sh.at[pl.ds(s * _RPT, _RPT)],
                    out_hbm.at[c, pl.ds(s * _RPT, _RPT)])


def _seg_sum(p, src2d, dst2d, ncols, cpt):
    mesh = plsc.VectorSubcoreMesh(core_axis_name="c", subcore_axis_name="s")
    zeros = jnp.zeros((_RPT, ncols), jnp.float32)
    kfn = pl.kernel(
        functools.partial(_seg_sum_body, ncols, cpt),
        out_type=jax.ShapeDtypeStruct((_NC, _NPAD, ncols), jnp.float32),
        mesh=mesh,
        compiler_params=pltpu.CompilerParams(use_tc_tiling_on_sc=False),
        scratch_types=[
            pltpu.VMEM((cpt, _CHUNK), jnp.int32),
            pltpu.VMEM((cpt, _CHUNK), jnp.int32),
            pltpu.VMEM((_CHUNK, ncols), jnp.float32),
            pltpu.VMEM_SHARED((_NPAD, ncols), jnp.float32),
        ])
    return kfn(p, src2d, dst2d, zeros)


def _deg_body(cpt, dst_hbm, z_hbm, out_hbm, dst_v, ones_v, deg_sh):
    c = lax.axis_index("c")
    s = lax.axis_index("s")
    wid = s * _NC + c
    one = jnp.ones((16,), jnp.float32)

    @pl.loop(0, _CHUNK)
    def _(r):
        ones_v[r, :] = one

    pltpu.sync_copy(z_hbm, deg_sh.at[pl.ds(s * _RPT, _RPT)])
    pltpu.sync_copy(dst_hbm.at[pl.ds(wid * cpt, cpt)], dst_v)
    plsc.subcore_barrier()

    @pl.loop(0, cpt)
    def _(j):
        pltpu.sync_copy(ones_v, deg_sh.at[dst_v.at[j]], add=True)

    plsc.subcore_barrier()
    pltpu.sync_copy(deg_sh.at[pl.ds(s * _RPT, _RPT)],
                    out_hbm.at[c, pl.ds(s * _RPT, _RPT)])


def _deg(dst2d, cpt):
    mesh = plsc.VectorSubcoreMesh(core_axis_name="c", subcore_axis_name="s")
    zeros = jnp.zeros((_RPT, 16), jnp.float32)
    kfn = pl.kernel(
        functools.partial(_deg_body, cpt),
        out_type=jax.ShapeDtypeStruct((_NC, _NPAD, 16), jnp.float32),
        mesh=mesh,
        compiler_params=pltpu.CompilerParams(use_tc_tiling_on_sc=False),
        scratch_types=[
            pltpu.VMEM((cpt, _CHUNK), jnp.int32),
            pltpu.VMEM((_CHUNK, 16), jnp.float32),
            pltpu.VMEM_SHARED((_NPAD, 16), jnp.float32),
        ])
    return kfn(dst2d, zeros)


def _a0_body(x_ref, wn_ref, o_ref):
    o_ref[...] = jnp.dot(x_ref[...], wn_ref[...],
                         preferred_element_type=jnp.float32)


def _c0_body(x_ref, ws, b, a0, a1, d0, d1, wn, h_ref, p_ref, inv_ref):
    inv = 1.0 / jnp.maximum(d0[...] + d1[...], 1.0)
    agg = (a0[...] + a1[...]) * inv
    h = jnp.tanh(jnp.dot(x_ref[...], ws[...],
                         preferred_element_type=jnp.float32) + agg + b[...])
    h_ref[...] = h
    p_ref[...] = jnp.dot(h, wn[...], preferred_element_type=jnp.float32)
    inv_ref[...] = inv


def _c_mid_body(h_in, ws, b, a0, a1, inv_ref_in, wn, h_ref, p_ref):
    inv = inv_ref_in[...]
    agg = (a0[...] + a1[...]) * inv
    h = jnp.tanh(jnp.dot(h_in[...], ws[...],
                         preferred_element_type=jnp.float32) + agg + b[...])
    h_ref[...] = h
    p_ref[...] = jnp.dot(h, wn[...], preferred_element_type=jnp.float32)


def _c_last_body(h_in, ws, b, a0, a1, inv_ref_in, h_ref):
    inv = inv_ref_in[...]
    agg = (a0[...] + a1[...]) * inv
    h_ref[...] = jnp.tanh(jnp.dot(h_in[...], ws[...],
                                  preferred_element_type=jnp.float32)
                          + agg + b[...])


def _head_body(h1, h2, h3, h4, w1, b1, w2, b2, o_ref):
    cs = jnp.concatenate([h1[...], h2[...], h3[...], h4[...]], axis=1)
    z = jnp.concatenate([cs[:500], cs[500:1000]], axis=1)
    z = jnp.maximum(jnp.dot(z, w1[...], preferred_element_type=jnp.float32)
                    + b1[...], 0.0)
    o_ref[...] = jax.nn.sigmoid(
        jnp.dot(z, w2[...], preferred_element_type=jnp.float32) + b2[...])


def _tc(body, out_shape, *args):
    return pl.pallas_call(body, out_shape=out_shape)(*args)


def kernel(x, edge_index, nlabel, Wn0, Ws0, b0, Wn1, Ws1, b1, Wn2, Ws2, b2,
           Wn3, Ws3, b3, lin1_W, lin1_b, lin2_W, lin2_b):
    n = x.shape[0]
    e = edge_index.shape[1]
    d_mid = Wn0.shape[1]
    cpt = -(-e // (_NW * _CHUNK))
    cpt = (cpt + 7) // 8 * 8
    epad = _NW * cpt * _CHUNK
    src = jnp.concatenate(
        [edge_index[0], jnp.zeros((epad - e,), jnp.int32)]).reshape(-1, _CHUNK)
    dst = jnp.concatenate(
        [edge_index[1],
         jnp.full((epad - e,), n, jnp.int32)]).reshape(-1, _CHUNK)

    f32 = jnp.float32
    hshape = jax.ShapeDtypeStruct((n, d_mid), f32)
    ishape = jax.ShapeDtypeStruct((n, 1), f32)

    degp = _deg(dst, cpt)
    p0 = _tc(_a0_body, hshape, x, Wn0)
    agg0 = _seg_sum(p0, src, dst, d_mid, cpt)
    h1, p1, inv = _tc(
        _c0_body, [hshape, hshape, ishape],
        x, Ws0, b0.reshape(1, -1), agg0[0, :n], agg0[1, :n],
        degp[0, :n, 0:1], degp[1, :n, 0:1], Wn1)
    agg1 = _seg_sum(p1, src, dst, d_mid, cpt)
    h2, p2 = _tc(
        _c_mid_body, [hshape, hshape],
        h1, Ws1, b1.reshape(1, -1), agg1[0, :n], agg1[1, :n], inv, Wn2)
    agg2 = _seg_sum(p2, src, dst, d_mid, cpt)
    h3, p3 = _tc(
        _c_mid_body, [hshape, hshape],
        h2, Ws2, b2.reshape(1, -1), agg2[0, :n], agg2[1, :n], inv, Wn3)
    agg3 = _seg_sum(p3, src, dst, d_mid, cpt)
    h4 = _tc(
        _c_last_body, hshape,
        h3, Ws3, b3.reshape(1, -1), agg3[0, :n], agg3[1, :n], inv)

    cs = jnp.concatenate([h1, h2, h3, h4], axis=1)
    out = _tc(
        _head_body, jax.ShapeDtypeStruct((500, 1), f32),
        h1[:1000], h2[:1000], h3[:1000], h4[:1000],
        lin1_W, lin1_b.reshape(1, -1), lin2_W, lin2_b.reshape(1, -1))
    return (out[:, 0], 0.0, cs)


# speedup vs baseline: 7.9970x; 1.2527x over previous
import functools

import jax
import jax.numpy as jnp
from jax import lax
from jax.experimental import pallas as pl
from jax.experimental.pallas import tpu as pltpu
from jax.experimental.pallas import tpu_sc as plsc

_NC = 2
_NS = 16
_NW = _NC * _NS
_CHUNK = 128
_NBUF = 8
_NPAD = 10240
_RPT = _NPAD // _NS


def _seg_sum_body(ncols, cpt, p_hbm, src_hbm, dst_hbm, z_hbm, out_hbm,
                  src_v, dst_v, rows_v, agg_sh, *gsems):
    c = lax.axis_index("c")
    s = lax.axis_index("s")
    wid = s * _NC + c
    pltpu.sync_copy(z_hbm, agg_sh.at[pl.ds(s * _RPT, _RPT)])
    pltpu.sync_copy(src_hbm.at[pl.ds(wid * cpt, cpt)], src_v)
    pltpu.sync_copy(dst_hbm.at[pl.ds(wid * cpt, cpt)], dst_v)
    plsc.subcore_barrier()

    for b in range(_NBUF):
        pltpu.async_copy(p_hbm.at[src_v.at[b]],
                         rows_v.at[pl.ds(b * _CHUNK, _CHUNK)], gsems[b])

    @pl.loop(0, cpt, step=_NBUF)
    def _(j0):
        for b in range(_NBUF):
            j = j0 + b
            slot = rows_v.at[pl.ds(b * _CHUNK, _CHUNK)]
            pltpu.make_async_copy(p_hbm.at[src_v.at[j]], slot,
                                  gsems[b]).wait()
            pltpu.sync_copy(slot, agg_sh.at[dst_v.at[j]], add=True)

            @pl.when(j + _NBUF < cpt)
            def _():
                pltpu.async_copy(p_hbm.at[src_v.at[j + _NBUF]], slot,
                                 gsems[b])

    plsc.subcore_barrier()
    pltpu.sync_copy(agg_sh.at[pl.ds(s * _RPT, _RPT)],
                    out_hbm.at[c, pl.ds(s * _RPT, _RPT)])


def _seg_sum(p, src2d, dst2d, ncols, cpt):
    mesh = plsc.VectorSubcoreMesh(core_axis_name="c", subcore_axis_name="s")
    zeros = jnp.zeros((_RPT, ncols), jnp.float32)
    kfn = pl.kernel(
        functools.partial(_seg_sum_body, ncols, cpt),
        out_type=jax.ShapeDtypeStruct((_NC, _NPAD, ncols), jnp.float32),
        mesh=mesh,
        compiler_params=pltpu.CompilerParams(use_tc_tiling_on_sc=False),
        scratch_types=[
            pltpu.VMEM((cpt, _CHUNK), jnp.int32),
            pltpu.VMEM((cpt, _CHUNK), jnp.int32),
            pltpu.VMEM((_NBUF * _CHUNK, ncols), jnp.float32),
            pltpu.VMEM_SHARED((_NPAD, ncols), jnp.float32),
        ] + [pltpu.SemaphoreType.DMA] * _NBUF)
    return kfn(p, src2d, dst2d, zeros)


def _deg_body(cpt, dst_hbm, z_hbm, out_hbm, dst_v, ones_v, deg_sh):
    c = lax.axis_index("c")
    s = lax.axis_index("s")
    wid = s * _NC + c
    one = jnp.ones((16,), jnp.float32)

    @pl.loop(0, _CHUNK)
    def _(r):
        ones_v[r, :] = one

    pltpu.sync_copy(z_hbm, deg_sh.at[pl.ds(s * _RPT, _RPT)])
    pltpu.sync_copy(dst_hbm.at[pl.ds(wid * cpt, cpt)], dst_v)
    plsc.subcore_barrier()

    @pl.loop(0, cpt)
    def _(j):
        pltpu.sync_copy(ones_v, deg_sh.at[dst_v.at[j]], add=True)

    plsc.subcore_barrier()
    pltpu.sync_copy(deg_sh.at[pl.ds(s * _RPT, _RPT)],
                    out_hbm.at[c, pl.ds(s * _RPT, _RPT)])


def _deg(dst2d, cpt):
    mesh = plsc.VectorSubcoreMesh(core_axis_name="c", subcore_axis_name="s")
    zeros = jnp.zeros((_RPT, 16), jnp.float32)
    kfn = pl.kernel(
        functools.partial(_deg_body, cpt),
        out_type=jax.ShapeDtypeStruct((_NC, _NPAD, 16), jnp.float32),
        mesh=mesh,
        compiler_params=pltpu.CompilerParams(use_tc_tiling_on_sc=False),
        scratch_types=[
            pltpu.VMEM((cpt, _CHUNK), jnp.int32),
            pltpu.VMEM((_CHUNK, 16), jnp.float32),
            pltpu.VMEM_SHARED((_NPAD, 16), jnp.float32),
        ])
    return kfn(dst2d, zeros)


def _a0_body(x_ref, wn_ref, o_ref):
    o_ref[...] = jnp.dot(x_ref[...], wn_ref[...],
                         preferred_element_type=jnp.float32)


def _c0_body(x_ref, ws, b, a0, a1, d0, d1, wn, h_ref, p_ref, inv_ref):
    inv = 1.0 / jnp.maximum(d0[...] + d1[...], 1.0)
    agg = (a0[...] + a1[...]) * inv
    h = jnp.tanh(jnp.dot(x_ref[...], ws[...],
                         preferred_element_type=jnp.float32) + agg + b[...])
    h_ref[...] = h
    p_ref[...] = jnp.dot(h, wn[...], preferred_element_type=jnp.float32)
    inv_ref[...] = inv


def _c_mid_body(h_in, ws, b, a0, a1, inv_ref_in, wn, h_ref, p_ref):
    inv = inv_ref_in[...]
    agg = (a0[...] + a1[...]) * inv
    h = jnp.tanh(jnp.dot(h_in[...], ws[...],
                         preferred_element_type=jnp.float32) + agg + b[...])
    h_ref[...] = h
    p_ref[...] = jnp.dot(h, wn[...], preferred_element_type=jnp.float32)


def _c_last_body(h_in, ws, b, a0, a1, inv_ref_in, h_ref):
    inv = inv_ref_in[...]
    agg = (a0[...] + a1[...]) * inv
    h_ref[...] = jnp.tanh(jnp.dot(h_in[...], ws[...],
                                  preferred_element_type=jnp.float32)
                          + agg + b[...])


def _head_body(h1, h2, h3, h4, w1, b1, w2, b2, o_ref):
    cs = jnp.concatenate([h1[...], h2[...], h3[...], h4[...]], axis=1)
    z = jnp.concatenate([cs[:500], cs[500:1000]], axis=1)
    z = jnp.maximum(jnp.dot(z, w1[...], preferred_element_type=jnp.float32)
                    + b1[...], 0.0)
    o_ref[...] = jax.nn.sigmoid(
        jnp.dot(z, w2[...], preferred_element_type=jnp.float32) + b2[...])


def _tc(body, out_shape, *args):
    return pl.pallas_call(body, out_shape=out_shape)(*args)


def kernel(x, edge_index, nlabel, Wn0, Ws0, b0, Wn1, Ws1, b1, Wn2, Ws2, b2,
           Wn3, Ws3, b3, lin1_W, lin1_b, lin2_W, lin2_b):
    n = x.shape[0]
    e = edge_index.shape[1]
    d_mid = Wn0.shape[1]
    cpt = -(-e // (_NW * _CHUNK))
    cpt = (cpt + 7) // 8 * 8
    epad = _NW * cpt * _CHUNK
    src = jnp.concatenate(
        [edge_index[0], jnp.zeros((epad - e,), jnp.int32)]).reshape(-1, _CHUNK)
    dst = jnp.concatenate(
        [edge_index[1],
         jnp.full((epad - e,), n, jnp.int32)]).reshape(-1, _CHUNK)

    f32 = jnp.float32
    hshape = jax.ShapeDtypeStruct((n, d_mid), f32)
    ishape = jax.ShapeDtypeStruct((n, 1), f32)

    degp = _deg(dst, cpt)
    p0 = _tc(_a0_body, hshape, x, Wn0)
    agg0 = _seg_sum(p0, src, dst, d_mid, cpt)
    h1, p1, inv = _tc(
        _c0_body, [hshape, hshape, ishape],
        x, Ws0, b0.reshape(1, -1), agg0[0, :n], agg0[1, :n],
        degp[0, :n, 0:1], degp[1, :n, 0:1], Wn1)
    agg1 = _seg_sum(p1, src, dst, d_mid, cpt)
    h2, p2 = _tc(
        _c_mid_body, [hshape, hshape],
        h1, Ws1, b1.reshape(1, -1), agg1[0, :n], agg1[1, :n], inv, Wn2)
    agg2 = _seg_sum(p2, src, dst, d_mid, cpt)
    h3, p3 = _tc(
        _c_mid_body, [hshape, hshape],
        h2, Ws2, b2.reshape(1, -1), agg2[0, :n], agg2[1, :n], inv, Wn3)
    agg3 = _seg_sum(p3, src, dst, d_mid, cpt)
    h4 = _tc(
        _c_last_body, hshape,
        h3, Ws3, b3.reshape(1, -1), agg3[0, :n], agg3[1, :n], inv)

    cs = jnp.concatenate([h1, h2, h3, h4], axis=1)
    out = _tc(
        _head_body, jax.ShapeDtypeStruct((500, 1), f32),
        h1[:1000], h2[:1000], h3[:1000], h4[:1000],
        lin1_W, lin1_b.reshape(1, -1), lin2_W, lin2_b.reshape(1, -1))
    return (out[:, 0], 0.0, cs)


# speedup vs baseline: 15.3839x; 1.9237x over previous
import functools

import jax
import jax.numpy as jnp
from jax import lax
from jax.experimental import pallas as pl
from jax.experimental.pallas import tpu as pltpu
from jax.experimental.pallas import tpu_sc as plsc

_NC = 2
_NS = 16
_NW = _NC * _NS
_CHUNK = 128
_NBUF = 8
_NPAD = 10240
_RPT = _NPAD // _NS


def _seg_sum_body(ncols, cpt, p_hbm, src_hbm, dst_hbm, z_hbm, out_hbm,
                  src_v, dst_v, rows_v, agg_sh, p_sh, *gsems):
    c = lax.axis_index("c")
    s = lax.axis_index("s")
    wid = s * _NC + c
    pltpu.sync_copy(z_hbm, agg_sh.at[pl.ds(s * _RPT, _RPT)])
    pltpu.sync_copy(p_hbm.at[pl.ds(s * _RPT, _RPT)],
                    p_sh.at[pl.ds(s * _RPT, _RPT)])
    pltpu.sync_copy(src_hbm.at[pl.ds(wid * cpt, cpt)], src_v)
    pltpu.sync_copy(dst_hbm.at[pl.ds(wid * cpt, cpt)], dst_v)
    plsc.subcore_barrier()

    for b in range(_NBUF):
        pltpu.async_copy(p_sh.at[src_v.at[b]],
                         rows_v.at[pl.ds(b * _CHUNK, _CHUNK)], gsems[b])

    @pl.loop(0, cpt, step=_NBUF)
    def _(j0):
        for b in range(_NBUF):
            j = j0 + b
            slot = rows_v.at[pl.ds(b * _CHUNK, _CHUNK)]
            pltpu.make_async_copy(p_sh.at[src_v.at[j]], slot,
                                  gsems[b]).wait()
            pltpu.sync_copy(slot, agg_sh.at[dst_v.at[j]], add=True)

            @pl.when(j + _NBUF < cpt)
            def _():
                pltpu.async_copy(p_sh.at[src_v.at[j + _NBUF]], slot,
                                 gsems[b])

    plsc.subcore_barrier()
    pltpu.sync_copy(agg_sh.at[pl.ds(s * _RPT, _RPT)],
                    out_hbm.at[c, pl.ds(s * _RPT, _RPT)])


def _seg_sum(p, src2d, dst2d, ncols, cpt):
    mesh = plsc.VectorSubcoreMesh(core_axis_name="c", subcore_axis_name="s")
    zeros = jnp.zeros((_RPT, ncols), jnp.float32)
    kfn = pl.kernel(
        functools.partial(_seg_sum_body, ncols, cpt),
        out_type=jax.ShapeDtypeStruct((_NC, _NPAD, ncols), jnp.float32),
        mesh=mesh,
        compiler_params=pltpu.CompilerParams(use_tc_tiling_on_sc=False),
        scratch_types=[
            pltpu.VMEM((cpt, _CHUNK), jnp.int32),
            pltpu.VMEM((cpt, _CHUNK), jnp.int32),
            pltpu.VMEM((_NBUF * _CHUNK, ncols), jnp.float32),
            pltpu.VMEM_SHARED((_NPAD, ncols), jnp.float32),
            pltpu.VMEM_SHARED((_NPAD, ncols), jnp.float32),
        ] + [pltpu.SemaphoreType.DMA] * _NBUF)
    return kfn(p, src2d, dst2d, zeros)


def _deg_body(cpt, dst_hbm, z_hbm, out_hbm, dst_v, ones_v, deg_sh):
    c = lax.axis_index("c")
    s = lax.axis_index("s")
    wid = s * _NC + c
    one = jnp.ones((16,), jnp.float32)

    @pl.loop(0, _CHUNK)
    def _(r):
        ones_v[r, :] = one

    pltpu.sync_copy(z_hbm, deg_sh.at[pl.ds(s * _RPT, _RPT)])
    pltpu.sync_copy(dst_hbm.at[pl.ds(wid * cpt, cpt)], dst_v)
    plsc.subcore_barrier()

    @pl.loop(0, cpt)
    def _(j):
        pltpu.sync_copy(ones_v, deg_sh.at[dst_v.at[j]], add=True)

    plsc.subcore_barrier()
    pltpu.sync_copy(deg_sh.at[pl.ds(s * _RPT, _RPT)],
                    out_hbm.at[c, pl.ds(s * _RPT, _RPT)])


def _deg(dst2d, cpt):
    mesh = plsc.VectorSubcoreMesh(core_axis_name="c", subcore_axis_name="s")
    zeros = jnp.zeros((_RPT, 16), jnp.float32)
    kfn = pl.kernel(
        functools.partial(_deg_body, cpt),
        out_type=jax.ShapeDtypeStruct((_NC, _NPAD, 16), jnp.float32),
        mesh=mesh,
        compiler_params=pltpu.CompilerParams(use_tc_tiling_on_sc=False),
        scratch_types=[
            pltpu.VMEM((cpt, _CHUNK), jnp.int32),
            pltpu.VMEM((_CHUNK, 16), jnp.float32),
            pltpu.VMEM_SHARED((_NPAD, 16), jnp.float32),
        ])
    return kfn(dst2d, zeros)


def _a0_body(x_ref, wn_ref, o_ref):
    o_ref[...] = jnp.dot(x_ref[...], wn_ref[...],
                         preferred_element_type=jnp.float32)


def _c0_body(x_ref, ws, b, a0, a1, d0, d1, wn, h_ref, p_ref, inv_ref):
    inv = 1.0 / jnp.maximum(d0[...] + d1[...], 1.0)
    agg = (a0[...] + a1[...]) * inv
    h = jnp.tanh(jnp.dot(x_ref[...], ws[...],
                         preferred_element_type=jnp.float32) + agg + b[...])
    h_ref[...] = h
    p_ref[...] = jnp.dot(h, wn[...], preferred_element_type=jnp.float32)
    inv_ref[...] = inv


def _c_mid_body(h_in, ws, b, a0, a1, inv_ref_in, wn, h_ref, p_ref):
    inv = inv_ref_in[...]
    agg = (a0[...] + a1[...]) * inv
    h = jnp.tanh(jnp.dot(h_in[...], ws[...],
                         preferred_element_type=jnp.float32) + agg + b[...])
    h_ref[...] = h
    p_ref[...] = jnp.dot(h, wn[...], preferred_element_type=jnp.float32)


def _c_last_body(h_in, ws, b, a0, a1, inv_ref_in, h_ref):
    inv = inv_ref_in[...]
    agg = (a0[...] + a1[...]) * inv
    h_ref[...] = jnp.tanh(jnp.dot(h_in[...], ws[...],
                                  preferred_element_type=jnp.float32)
                          + agg + b[...])


def _head_body(h1, h2, h3, h4, w1, b1, w2, b2, o_ref):
    cs = jnp.concatenate([h1[...], h2[...], h3[...], h4[...]], axis=1)
    z = jnp.concatenate([cs[:500], cs[500:1000]], axis=1)
    z = jnp.maximum(jnp.dot(z, w1[...], preferred_element_type=jnp.float32)
                    + b1[...], 0.0)
    o_ref[...] = jax.nn.sigmoid(
        jnp.dot(z, w2[...], preferred_element_type=jnp.float32) + b2[...])


def _tc(body, out_shape, *args):
    return pl.pallas_call(body, out_shape=out_shape)(*args)


def kernel(x, edge_index, nlabel, Wn0, Ws0, b0, Wn1, Ws1, b1, Wn2, Ws2, b2,
           Wn3, Ws3, b3, lin1_W, lin1_b, lin2_W, lin2_b):
    n = x.shape[0]
    e = edge_index.shape[1]
    d_mid = Wn0.shape[1]
    cpt = -(-e // (_NW * _CHUNK))
    cpt = (cpt + 7) // 8 * 8
    epad = _NW * cpt * _CHUNK
    src = jnp.concatenate(
        [edge_index[0], jnp.zeros((epad - e,), jnp.int32)]).reshape(-1, _CHUNK)
    dst = jnp.concatenate(
        [edge_index[1],
         jnp.full((epad - e,), n, jnp.int32)]).reshape(-1, _CHUNK)

    f32 = jnp.float32
    xp = jnp.concatenate([x, jnp.zeros((_NPAD - n, x.shape[1]), f32)])
    hshape = jax.ShapeDtypeStruct((_NPAD, d_mid), f32)
    ishape = jax.ShapeDtypeStruct((_NPAD, 1), f32)

    degp = _deg(dst, cpt)
    p0 = _tc(_a0_body, hshape, xp, Wn0)
    agg0 = _seg_sum(p0, src, dst, d_mid, cpt)
    h1, p1, inv = _tc(
        _c0_body, [hshape, hshape, ishape],
        xp, Ws0, b0.reshape(1, -1), agg0[0], agg0[1],
        degp[0, :, 0:1], degp[1, :, 0:1], Wn1)
    agg1 = _seg_sum(p1, src, dst, d_mid, cpt)
    h2, p2 = _tc(
        _c_mid_body, [hshape, hshape],
        h1, Ws1, b1.reshape(1, -1), agg1[0], agg1[1], inv, Wn2)
    agg2 = _seg_sum(p2, src, dst, d_mid, cpt)
    h3, p3 = _tc(
        _c_mid_body, [hshape, hshape],
        h2, Ws2, b2.reshape(1, -1), agg2[0], agg2[1], inv, Wn3)
    agg3 = _seg_sum(p3, src, dst, d_mid, cpt)
    h4 = _tc(
        _c_last_body, hshape,
        h3, Ws3, b3.reshape(1, -1), agg3[0], agg3[1], inv)

    cs = jnp.concatenate([h1[:n], h2[:n], h3[:n], h4[:n]], axis=1)
    out = _tc(
        _head_body, jax.ShapeDtypeStruct((500, 1), f32),
        h1[:1000], h2[:1000], h3[:1000], h4[:1000],
        lin1_W, lin1_b.reshape(1, -1), lin2_W, lin2_b.reshape(1, -1))
    return (out[:, 0], 0.0, cs)


# speedup vs baseline: 16.4899x; 1.0719x over previous
import functools

import jax
import jax.numpy as jnp
from jax import lax
from jax.experimental import pallas as pl
from jax.experimental.pallas import tpu as pltpu
from jax.experimental.pallas import tpu_sc as plsc

_NC = 2
_NS = 16
_NW = _NC * _NS
_CHUNK = 128
_NBUF = 8
_NPAD = 10240
_RPT = _NPAD // _NS


def _seg_sum_body(ncols, cpt, with_deg, *refs):
    if with_deg:
        (p_hbm, src_hbm, dst_hbm, z_hbm, z16_hbm, out_hbm, deg_hbm,
         src_v, dst_v, rows_v, ones_v, agg_sh, p_sh, deg_sh, *gsems) = refs
    else:
        (p_hbm, src_hbm, dst_hbm, z_hbm, out_hbm,
         src_v, dst_v, rows_v, agg_sh, p_sh, *gsems) = refs
    c = lax.axis_index("c")
    s = lax.axis_index("s")
    wid = s * _NC + c
    pltpu.sync_copy(z_hbm, agg_sh.at[pl.ds(s * _RPT, _RPT)])
    pltpu.sync_copy(p_hbm.at[pl.ds(s * _RPT, _RPT)],
                    p_sh.at[pl.ds(s * _RPT, _RPT)])
    pltpu.sync_copy(src_hbm.at[pl.ds(wid * cpt, cpt)], src_v)
    pltpu.sync_copy(dst_hbm.at[pl.ds(wid * cpt, cpt)], dst_v)
    if with_deg:
        pltpu.sync_copy(z16_hbm, deg_sh.at[pl.ds(s * _RPT, _RPT)])
        one = jnp.ones((16,), jnp.float32)

        @pl.loop(0, _CHUNK)
        def _(r):
            ones_v[r, :] = one
    plsc.subcore_barrier()

    for b in range(_NBUF):
        pltpu.async_copy(p_sh.at[src_v.at[b]],
                         rows_v.at[pl.ds(b * _CHUNK, _CHUNK)], gsems[b])

    @pl.loop(0, cpt, step=_NBUF)
    def _(j0):
        for b in range(_NBUF):
            j = j0 + b
            slot = rows_v.at[pl.ds(b * _CHUNK, _CHUNK)]
            pltpu.make_async_copy(p_sh.at[src_v.at[j]], slot,
                                  gsems[b]).wait()
            pltpu.sync_copy(slot, agg_sh.at[dst_v.at[j]], add=True)
            if with_deg:
                pltpu.sync_copy(ones_v, deg_sh.at[dst_v.at[j]], add=True)

            @pl.when(j + _NBUF < cpt)
            def _():
                pltpu.async_copy(p_sh.at[src_v.at[j + _NBUF]], slot,
                                 gsems[b])

    plsc.subcore_barrier()
    pltpu.sync_copy(agg_sh.at[pl.ds(s * _RPT, _RPT)],
                    out_hbm.at[c, pl.ds(s * _RPT, _RPT)])
    if with_deg:
        pltpu.sync_copy(deg_sh.at[pl.ds(s * _RPT, _RPT)],
                        deg_hbm.at[c, pl.ds(s * _RPT, _RPT)])


def _seg_sum(p, src2d, dst2d, ncols, cpt, with_deg=False):
    mesh = plsc.VectorSubcoreMesh(core_axis_name="c", subcore_axis_name="s")
    zeros = jnp.zeros((_RPT, ncols), jnp.float32)
    out_t = jax.ShapeDtypeStruct((_NC, _NPAD, ncols), jnp.float32)
    scratch = [
        pltpu.VMEM((cpt, _CHUNK), jnp.int32),
        pltpu.VMEM((cpt, _CHUNK), jnp.int32),
        pltpu.VMEM((_NBUF * _CHUNK, ncols), jnp.float32),
    ]
    shared = [
        pltpu.VMEM_SHARED((_NPAD, ncols), jnp.float32),
        pltpu.VMEM_SHARED((_NPAD, ncols), jnp.float32),
    ]
    if with_deg:
        out_t = [out_t, jax.ShapeDtypeStruct((_NC, _NPAD, 16), jnp.float32)]
        scratch = scratch + [pltpu.VMEM((_CHUNK, 16), jnp.float32)]
        shared = shared + [pltpu.VMEM_SHARED((_NPAD, 16), jnp.float32)]
    kfn = pl.kernel(
        functools.partial(_seg_sum_body, ncols, cpt, with_deg),
        out_type=out_t,
        mesh=mesh,
        compiler_params=pltpu.CompilerParams(use_tc_tiling_on_sc=False),
        scratch_types=scratch + shared + [pltpu.SemaphoreType.DMA] * _NBUF)
    if with_deg:
        z16 = jnp.zeros((_RPT, 16), jnp.float32)
        return kfn(p, src2d, dst2d, zeros, z16)
    return kfn(p, src2d, dst2d, zeros)


def _a0_body(x_ref, wn_ref, o_ref):
    o_ref[...] = jnp.dot(x_ref[...], wn_ref[...],
                         preferred_element_type=jnp.float32)


def _c0_body(x_ref, ws, b, agg2, deg2, wn, h_ref, p_ref, inv_ref):
    deg = deg2[0, :, 0:1] + deg2[1, :, 0:1]
    inv = 1.0 / jnp.maximum(deg, 1.0)
    agg = (agg2[0] + agg2[1]) * inv
    h = jnp.tanh(jnp.dot(x_ref[...], ws[...],
                         preferred_element_type=jnp.float32) + agg + b[...])
    h_ref[...] = h
    p_ref[...] = jnp.dot(h, wn[...], preferred_element_type=jnp.float32)
    inv_ref[...] = inv


def _c_mid_body(h_in, ws, b, agg2, inv_ref_in, wn, h_ref, p_ref):
    inv = inv_ref_in[...]
    agg = (agg2[0] + agg2[1]) * inv
    h = jnp.tanh(jnp.dot(h_in[...], ws[...],
                         preferred_element_type=jnp.float32) + agg + b[...])
    h_ref[...] = h
    p_ref[...] = jnp.dot(h, wn[...], preferred_element_type=jnp.float32)


def _c_last_body(h_in, ws, b, agg2, inv_ref_in, h_ref):
    inv = inv_ref_in[...]
    agg = (agg2[0] + agg2[1]) * inv
    h_ref[...] = jnp.tanh(jnp.dot(h_in[...], ws[...],
                                  preferred_element_type=jnp.float32)
                          + agg + b[...])


def _head_body(h1, h2, h3, h4, w1, b1, w2, b2, o_ref):
    cs = jnp.concatenate([h1[...], h2[...], h3[...], h4[...]], axis=1)
    z = jnp.concatenate([cs[:500], cs[500:1000]], axis=1)
    z = jnp.maximum(jnp.dot(z, w1[...], preferred_element_type=jnp.float32)
                    + b1[...], 0.0)
    o_ref[...] = jax.nn.sigmoid(
        jnp.dot(z, w2[...], preferred_element_type=jnp.float32) + b2[...])


def _tc(body, out_shape, *args):
    return pl.pallas_call(body, out_shape=out_shape)(*args)


def kernel(x, edge_index, nlabel, Wn0, Ws0, b0, Wn1, Ws1, b1, Wn2, Ws2, b2,
           Wn3, Ws3, b3, lin1_W, lin1_b, lin2_W, lin2_b):
    n = x.shape[0]
    e = edge_index.shape[1]
    d_mid = Wn0.shape[1]
    cpt = -(-e // (_NW * _CHUNK))
    cpt = (cpt + 7) // 8 * 8
    epad = _NW * cpt * _CHUNK
    src = jnp.concatenate(
        [edge_index[0], jnp.zeros((epad - e,), jnp.int32)]).reshape(-1, _CHUNK)
    dst = jnp.concatenate(
        [edge_index[1],
         jnp.full((epad - e,), n, jnp.int32)]).reshape(-1, _CHUNK)

    f32 = jnp.float32
    xp = jnp.concatenate([x, jnp.zeros((_NPAD - n, x.shape[1]), f32)])
    hshape = jax.ShapeDtypeStruct((_NPAD, d_mid), f32)
    ishape = jax.ShapeDtypeStruct((_NPAD, 1), f32)

    p0 = _tc(_a0_body, hshape, xp, Wn0)
    agg0, degp = _seg_sum(p0, src, dst, d_mid, cpt, with_deg=True)
    h1, p1, inv = _tc(
        _c0_body, [hshape, hshape, ishape],
        xp, Ws0, b0.reshape(1, -1), agg0, degp, Wn1)
    agg1 = _seg_sum(p1, src, dst, d_mid, cpt)
    h2, p2 = _tc(
        _c_mid_body, [hshape, hshape],
        h1, Ws1, b1.reshape(1, -1), agg1, inv, Wn2)
    agg2 = _seg_sum(p2, src, dst, d_mid, cpt)
    h3, p3 = _tc(
        _c_mid_body, [hshape, hshape],
        h2, Ws2, b2.reshape(1, -1), agg2, inv, Wn3)
    agg3 = _seg_sum(p3, src, dst, d_mid, cpt)
    h4 = _tc(
        _c_last_body, hshape,
        h3, Ws3, b3.reshape(1, -1), agg3, inv)

    cs = jnp.concatenate([h1[:n], h2[:n], h3[:n], h4[:n]], axis=1)
    out = _tc(
        _head_body, jax.ShapeDtypeStruct((500, 1), f32),
        h1[:1000], h2[:1000], h3[:1000], h4[:1000],
        lin1_W, lin1_b.reshape(1, -1), lin2_W, lin2_b.reshape(1, -1))
    return (out[:, 0], 0.0, cs)


# speedup vs baseline: 17.8275x; 1.0811x over previous
import functools

import jax
import jax.numpy as jnp
from jax import lax
from jax.experimental import pallas as pl
from jax.experimental.pallas import tpu as pltpu
from jax.experimental.pallas import tpu_sc as plsc

_NC = 2
_NS = 16
_NW = _NC * _NS
_CHUNK = 125
_NBUF = 8
_NPAD = 10240
_RPT = _NPAD // _NS


def _seg_sum_body(ncols, cpt, with_deg, *refs):
    if with_deg:
        (p_hbm, ei_hbm, z_hbm, z16_hbm, out_hbm, deg_hbm,
         src_v, dst_v, rows_v, ones_v, agg_sh, p_sh, deg_sh, *gsems) = refs
    else:
        (p_hbm, ei_hbm, z_hbm, out_hbm,
         src_v, dst_v, rows_v, agg_sh, p_sh, *gsems) = refs
    c = lax.axis_index("c")
    s = lax.axis_index("s")
    wid = s * _NC + c
    pltpu.sync_copy(z_hbm, agg_sh.at[pl.ds(s * _RPT, _RPT)])
    pltpu.sync_copy(p_hbm.at[pl.ds(s * _RPT, _RPT)],
                    p_sh.at[pl.ds(s * _RPT, _RPT)])
    pltpu.sync_copy(ei_hbm.at[0, pl.ds(wid * cpt, cpt)], src_v)
    pltpu.sync_copy(ei_hbm.at[1, pl.ds(wid * cpt, cpt)], dst_v)
    if with_deg:
        pltpu.sync_copy(z16_hbm, deg_sh.at[pl.ds(s * _RPT, _RPT)])
        one = jnp.ones((16,), jnp.float32)

        @pl.loop(0, _CHUNK)
        def _(r):
            ones_v[r, :] = one
    plsc.subcore_barrier()

    for b in range(_NBUF):
        pltpu.async_copy(p_sh.at[src_v.at[b]],
                         rows_v.at[pl.ds(b * 128, _CHUNK)], gsems[b])

    @pl.loop(0, cpt, step=_NBUF)
    def _(j0):
        for b in range(_NBUF):
            j = j0 + b
            slot = rows_v.at[pl.ds(b * 128, _CHUNK)]
            pltpu.make_async_copy(p_sh.at[src_v.at[j]], slot,
                                  gsems[b]).wait()
            pltpu.sync_copy(slot, agg_sh.at[dst_v.at[j]], add=True)
            if with_deg:
                pltpu.sync_copy(ones_v, deg_sh.at[dst_v.at[j]], add=True)

            @pl.when(j + _NBUF < cpt)
            def _():
                pltpu.async_copy(p_sh.at[src_v.at[j + _NBUF]], slot,
                                 gsems[b])

    plsc.subcore_barrier()
    pltpu.sync_copy(agg_sh.at[pl.ds(s * _RPT, _RPT)],
                    out_hbm.at[c, pl.ds(s * _RPT, _RPT)])
    if with_deg:
        pltpu.sync_copy(deg_sh.at[pl.ds(s * _RPT, _RPT)],
                        deg_hbm.at[c, pl.ds(s * _RPT, _RPT)])


def _seg_sum(p, ei3d, ncols, cpt, with_deg=False):
    mesh = plsc.VectorSubcoreMesh(core_axis_name="c", subcore_axis_name="s")
    zeros = jnp.zeros((_RPT, ncols), jnp.float32)
    out_t = jax.ShapeDtypeStruct((_NC, _NPAD, ncols), jnp.float32)
    scratch = [
        pltpu.VMEM((cpt, _CHUNK), jnp.int32),
        pltpu.VMEM((cpt, _CHUNK), jnp.int32),
        pltpu.VMEM((_NBUF * 128, ncols), jnp.float32),
    ]
    shared = [
        pltpu.VMEM_SHARED((_NPAD, ncols), jnp.float32),
        pltpu.VMEM_SHARED((_NPAD, ncols), jnp.float32),
    ]
    if with_deg:
        out_t = [out_t, jax.ShapeDtypeStruct((_NC, _NPAD, 16), jnp.float32)]
        scratch = scratch + [pltpu.VMEM((_CHUNK, 16), jnp.float32)]
        shared = shared + [pltpu.VMEM_SHARED((_NPAD, 16), jnp.float32)]
    kfn = pl.kernel(
        functools.partial(_seg_sum_body, ncols, cpt, with_deg),
        out_type=out_t,
        mesh=mesh,
        compiler_params=pltpu.CompilerParams(use_tc_tiling_on_sc=False),
        scratch_types=scratch + shared + [pltpu.SemaphoreType.DMA] * _NBUF)
    if with_deg:
        z16 = jnp.zeros((_RPT, 16), jnp.float32)
        return kfn(p, ei3d, zeros, z16)
    return kfn(p, ei3d, zeros)


def _a0_body(n, x_ref, wn_ref, o_ref):
    o_ref[pl.ds(0, n)] = jnp.dot(x_ref[...], wn_ref[...],
                                 preferred_element_type=jnp.float32)


def _c0_body(n, x_ref, ws, b, agg2, deg2, wn, h_ref, p_ref, inv_ref):
    deg = deg2[0, 0:n, 0:1] + deg2[1, 0:n, 0:1]
    inv = 1.0 / jnp.maximum(deg, 1.0)
    agg = (agg2[0, 0:n] + agg2[1, 0:n]) * inv
    h = jnp.tanh(jnp.dot(x_ref[...], ws[...],
                         preferred_element_type=jnp.float32) + agg + b[...])
    h_ref[pl.ds(0, n)] = h
    p_ref[pl.ds(0, n)] = jnp.dot(h, wn[...],
                                 preferred_element_type=jnp.float32)
    inv_ref[pl.ds(0, n)] = inv


def _c_mid_body(n, h_in, ws, b, agg2, inv_ref_in, wn, h_ref, p_ref):
    inv = inv_ref_in[0:n]
    agg = (agg2[0, 0:n] + agg2[1, 0:n]) * inv
    h = jnp.tanh(jnp.dot(h_in[0:n], ws[...],
                         preferred_element_type=jnp.float32) + agg + b[...])
    h_ref[pl.ds(0, n)] = h
    p_ref[pl.ds(0, n)] = jnp.dot(h, wn[...],
                                 preferred_element_type=jnp.float32)


def _c_last_body(n, h_in, ws, b, agg2, inv_ref_in, h_ref):
    inv = inv_ref_in[0:n]
    agg = (agg2[0, 0:n] + agg2[1, 0:n]) * inv
    h_ref[pl.ds(0, n)] = jnp.tanh(jnp.dot(h_in[0:n], ws[...],
                                          preferred_element_type=jnp.float32)
                                  + agg + b[...])


def _head_body(h1, h2, h3, h4, w1, b1, w2, b2, o_ref):
    cs = jnp.concatenate([h1[...], h2[...], h3[...], h4[...]], axis=1)
    z = jnp.concatenate([cs[:500], cs[500:1000]], axis=1)
    z = jnp.maximum(jnp.dot(z, w1[...], preferred_element_type=jnp.float32)
                    + b1[...], 0.0)
    o_ref[...] = jax.nn.sigmoid(
        jnp.dot(z, w2[...], preferred_element_type=jnp.float32) + b2[...])


def _tc(body, out_shape, *args):
    return pl.pallas_call(body, out_shape=out_shape)(*args)


def kernel(x, edge_index, nlabel, Wn0, Ws0, b0, Wn1, Ws1, b1, Wn2, Ws2, b2,
           Wn3, Ws3, b3, lin1_W, lin1_b, lin2_W, lin2_b):
    n = x.shape[0]
    e = edge_index.shape[1]
    d_mid = Wn0.shape[1]
    nchunks = e // _CHUNK
    cpt = nchunks // _NW
    ei3 = edge_index.reshape(2, nchunks, _CHUNK)

    f32 = jnp.float32
    hshape = jax.ShapeDtypeStruct((_NPAD, d_mid), f32)
    ishape = jax.ShapeDtypeStruct((_NPAD, 1), f32)

    p0 = _tc(functools.partial(_a0_body, n), hshape, x, Wn0)
    agg0, degp = _seg_sum(p0, ei3, d_mid, cpt, with_deg=True)
    h1, p1, inv = _tc(
        functools.partial(_c0_body, n), [hshape, hshape, ishape],
        x, Ws0, b0.reshape(1, -1), agg0, degp, Wn1)
    agg1 = _seg_sum(p1, ei3, d_mid, cpt)
    h2, p2 = _tc(
        functools.partial(_c_mid_body, n), [hshape, hshape],
        h1, Ws1, b1.reshape(1, -1), agg1, inv, Wn2)
    agg2 = _seg_sum(p2, ei3, d_mid, cpt)
    h3, p3 = _tc(
        functools.partial(_c_mid_body, n), [hshape, hshape],
        h2, Ws2, b2.reshape(1, -1), agg2, inv, Wn3)
    agg3 = _seg_sum(p3, ei3, d_mid, cpt)
    h4 = _tc(
        functools.partial(_c_last_body, n), hshape,
        h3, Ws3, b3.reshape(1, -1), agg3, inv)

    cs = jnp.concatenate([h1[:n], h2[:n], h3[:n], h4[:n]], axis=1)
    out = _tc(
        _head_body, jax.ShapeDtypeStruct((500, 1), f32),
        h1[:1000], h2[:1000], h3[:1000], h4[:1000],
        lin1_W, lin1_b.reshape(1, -1), lin2_W, lin2_b.reshape(1, -1))
    return (out[:, 0], 0.0, cs)
